# Initial kernel scaffold; baseline (speedup 1.0000x reference)
import jax
import jax.numpy as jnp
from jax.experimental import pallas as pl


def kernel(features, coords, instance_ids, W1, W2, W3, b3):
    raise NotImplementedError("write your pallas kernel here")


# speedup vs baseline: 1.6319x; 1.6319x over previous
import jax
import jax.numpy as jnp
from jax import lax
from jax.experimental import pallas as pl
from jax.experimental.pallas import tpu as pltpu
from jax.experimental.pallas import tpu_sc as plsc

N = 320000
D = 128
S = 10000
OUT = 32

NC = 2
NS = 16
SEGH = S // NC
TROWS = 5008
RPW = N // NS
BLK = 160
NBLK = RPW // BLK
IDXW = 80
NIDX = BLK // IDXW
G = 16
NGRP = RPW // G
CW = 16
CROWS = 632
FB = 80


def _sc_segment_sums(features, cones, instance_ids):
    mesh = plsc.VectorSubcoreMesh(core_axis_name="c", subcore_axis_name="s",
                                  num_cores=NC)

    f = pl.kernel(
        _sc_body,
        out_type=(
            jax.ShapeDtypeStruct((NC, TROWS, D), jnp.float32),
            jax.ShapeDtypeStruct((NC, CROWS, D), jnp.float32),
        ),
        mesh=mesh,
        scratch_types=(
            pltpu.VMEM((BLK, D), jnp.float32),
            pltpu.VMEM((BLK, CW), jnp.float32),
            pltpu.VMEM((RPW,), jnp.int32),
            pltpu.VMEM((NIDX, IDXW), jnp.int32),
            pltpu.VMEM((FB, D), jnp.float32),
            pltpu.VMEM((1, FB), jnp.int32),
            pltpu.VMEM((16,), jnp.float32),
            pltpu.VMEM_SHARED((TROWS, D), jnp.float32),
            pltpu.VMEM_SHARED((CROWS, D), jnp.float32),
        ),
    )
    return f(features, cones, instance_ids)


def _sc_body(feat_hbm, cones_hbm, ids_hbm, emb_out, c_out, fbuf, cbuf, ibig,
             sbuf, stage, idxbuf, accbuf, ftab, ctab):
    cid = lax.axis_index("c")
    sid = lax.axis_index("s")

    z16 = jnp.zeros((16,), jnp.float32)
    iota16 = lax.iota(jnp.int32, 16)
    base = cid * SEGH

    def zero_row(r, _):
        for k in range(D // 16):
            fbuf[r, pl.ds(k * 16, 16)] = z16
        return 0

    lax.fori_loop(0, BLK, zero_row, 0)
    for cch in range(FB // 16):
        idxbuf[0, pl.ds(cch * 16, 16)] = jnp.zeros((16,), jnp.int32)

    @pl.when(sid < NS - 1)
    def _():
        s0 = sid * 312
        pltpu.sync_copy(fbuf.at[pl.ds(0, 160)], ftab.at[pl.ds(s0, 160)])
        pltpu.sync_copy(fbuf.at[pl.ds(0, 152)], ftab.at[pl.ds(s0 + 160, 152)])
        pltpu.sync_copy(fbuf.at[pl.ds(0, 40)], ctab.at[pl.ds(sid * 40, 40)])

    @pl.when(sid == NS - 1)
    def _():
        pltpu.sync_copy(fbuf.at[pl.ds(0, 160)], ftab.at[pl.ds(4680, 160)])
        pltpu.sync_copy(fbuf.at[pl.ds(0, 160)], ftab.at[pl.ds(4840, 160)])
        pltpu.sync_copy(fbuf.at[pl.ds(0, 8)], ftab.at[pl.ds(5000, 8)])
        pltpu.sync_copy(fbuf.at[pl.ds(0, 32)], ctab.at[pl.ds(600, 32)])

    row_base = sid * RPW
    pltpu.sync_copy(ids_hbm.at[pl.ds(row_base, RPW)], ibig)

    plsc.subcore_barrier()

    def lower_bound(bound):
        def step(_, c):
            lo, hi = c
            mid = (lo + hi) >> 1
            h = ibig[pl.ds(mid * G, G)][0]
            take = h < bound
            lo = jnp.where(take, mid + 1, lo)
            hi = jnp.where(take, hi, mid)
            return lo, hi
        lo, _ = lax.fori_loop(0, 11, step, (jnp.int32(0), jnp.int32(NGRP)))
        return lo

    gb = lower_bound(base)
    ge = lower_bound(base + SEGH)
    blo = jnp.maximum(gb - 1, 0) * G // BLK
    bhi = jnp.minimum(jnp.int32(NBLK), (ge * G + BLK - 1) // BLK)

    prev0 = ibig[pl.ds(blo * BLK, G)][0]

    def clamp_loc(x):
        x = x - base
        return jnp.minimum(x + ((x >> 31) & 16384), SEGH)

    def park_run(cnt, lp, acc):
        slot = lp & 7
        accbuf[...] = acc

        def store_row(r, _):
            for s in range(8):
                ms = jnp.where(slot == s, 1.0, 0.0)
                stage[r, pl.ds(s * 16, 16)] = accbuf[...] * ms
            return 0

        lax.fori_loop(cnt, cnt + 1, store_row, 0)
        for cch in range(FB // 16):
            m = 1 - jnp.minimum((iota16 + cch * 16) ^ cnt, 1)
            iv = idxbuf[0, pl.ds(cch * 16, 16)]
            idxbuf[0, pl.ds(cch * 16, 16)] = iv + m * ((lp >> 3) - iv)

    def block_body(b, carry):
        row0 = row_base + b * BLK
        pltpu.sync_copy(feat_hbm.at[pl.ds(row0, BLK)], fbuf)
        pltpu.sync_copy(cones_hbm.at[pl.ds(row0, BLK)], cbuf)

        for j in range(NIDX):
            for c in range(IDXW // G):
                idv = ibig[pl.ds(b * BLK + j * IDXW + c * G, G)]
                sbuf[j, pl.ds(c * G, G)] = clamp_loc(idv)
        for j in range(NIDX):
            pltpu.sync_copy(fbuf.at[pl.ds(j * IDXW, IDXW)],
                            ftab.at[sbuf.at[j]], add=True)

        def grp_body(g, carry):
            acc, prev, cnt = carry
            idv = ibig[pl.ds(b * BLK + g * G, G)]
            for k in range(G):
                sgid = idv[k]
                changed = sgid != prev
                chi = changed.astype(jnp.int32)
                lp = clamp_loc(prev)

                @pl.when(changed)
                def _(cnt=cnt, lp=lp, acc=acc):
                    park_run(cnt, lp, acc)

                cnt2 = cnt + chi

                @pl.when(cnt2 == FB)
                def _():
                    pltpu.sync_copy(stage.at[pl.ds(0, FB)], ctab.at[idxbuf.at[0]], add=True)

                cnt = jnp.where(cnt2 == FB, 0, cnt2)
                chf = chi.astype(jnp.float32)
                acc = acc * (1.0 - chf) + cbuf[g * G + k, pl.ds(0, 16)]
                prev = sgid
            return acc, prev, cnt

        return lax.fori_loop(0, BLK // G, grp_body, carry)

    carry = (jnp.zeros((16,), jnp.float32), prev0, jnp.int32(0))
    acc, prev, cnt = lax.fori_loop(blo, bhi, block_body, carry)

    park_run(cnt, clamp_loc(prev), acc)

    def zero_tail(r, _):
        @pl.when(r > cnt)
        def _():
            for s in range(8):
                stage[r, pl.ds(s * 16, 16)] = z16
        return 0

    lax.fori_loop(0, FB, zero_tail, 0)
    pltpu.sync_copy(stage.at[pl.ds(0, FB)], ctab.at[idxbuf.at[0]], add=True)

    plsc.subcore_barrier()

    def copy_out(tab, out, s0, ln):
        pltpu.sync_copy(tab.at[pl.ds(s0, ln)], fbuf.at[pl.ds(0, ln)])
        pltpu.sync_copy(fbuf.at[pl.ds(0, ln)], out.at[cid, pl.ds(s0, ln)])

    @pl.when(sid < NS - 1)
    def _():
        copy_out(ftab, emb_out, sid * 312, 160)
        copy_out(ftab, emb_out, sid * 312 + 160, 152)
        copy_out(ctab, c_out, sid * 40, 40)

    @pl.when(sid == NS - 1)
    def _():
        copy_out(ftab, emb_out, 4680, 160)
        copy_out(ftab, emb_out, 4840, 160)
        copy_out(ftab, emb_out, 5000, 8)
        copy_out(ctab, c_out, 600, 32)


def _tc_finalize_body(emb_ref, c_ref, w1_ref, w2_ref, w3_ref, b3_ref,
                      emb_out, cent_out, mlp_out):
    e = emb_ref[...]
    c4 = c_ref[...]
    counts = c4[:, 3:4]
    denom = jnp.maximum(counts, 1.0)
    emb = e / denom
    cent_out[...] = c4 / denom
    emb_out[...] = emb
    h = jnp.maximum(jnp.dot(emb, w1_ref[...],
                            preferred_element_type=jnp.float32), 0.0)
    h = jnp.maximum(jnp.dot(h, w2_ref[...],
                            preferred_element_type=jnp.float32), 0.0)
    mlp_out[...] = jnp.dot(h, w3_ref[...],
                           preferred_element_type=jnp.float32) + b3_ref[...]


def _tc_finalize(emb_sums, c_sums, W1, W2, W3, b3):
    SB = 400
    grid = (S // SB,)
    return pl.pallas_call(
        _tc_finalize_body,
        grid=grid,
        in_specs=[
            pl.BlockSpec((SB, D), lambda i: (i, 0)),
            pl.BlockSpec((SB, CW), lambda i: (i, 0)),
            pl.BlockSpec((D, 64), lambda i: (0, 0)),
            pl.BlockSpec((64, 64), lambda i: (0, 0)),
            pl.BlockSpec((64, OUT), lambda i: (0, 0)),
            pl.BlockSpec((1, OUT), lambda i: (0, 0)),
        ],
        out_specs=[
            pl.BlockSpec((SB, D), lambda i: (i, 0)),
            pl.BlockSpec((SB, CW), lambda i: (i, 0)),
            pl.BlockSpec((SB, OUT), lambda i: (i, 0)),
        ],
        out_shape=[
            jax.ShapeDtypeStruct((S, D), jnp.float32),
            jax.ShapeDtypeStruct((S, CW), jnp.float32),
            jax.ShapeDtypeStruct((S, OUT), jnp.float32),
        ],
    )(emb_sums, c_sums, W1, W2, W3, b3)


def kernel(features, coords, instance_ids, W1, W2, W3, b3):
    ones = jnp.ones((N, 1), jnp.float32)
    pad = jnp.zeros((N, CW - 4), jnp.float32)
    cones = jnp.concatenate([coords, ones, pad], axis=1)

    parts_emb, parts_cpacked = _sc_segment_sums(features, cones, instance_ids)
    emb_sums = parts_emb[:, :SEGH, :].reshape(S, D)
    c_sums = parts_cpacked[:, :SEGH // 8, :].reshape(S, CW)
    emb, cent4, out = _tc_finalize(emb_sums, c_sums, W1, W2, W3,
                                   b3.reshape(1, OUT))
    return emb, cent4[:, :3], out
